# Initial kernel scaffold; baseline (speedup 1.0000x reference)
import jax
import jax.numpy as jnp
from jax.experimental import pallas as pl


def kernel(x, params):
    raise NotImplementedError("write your pallas kernel here")


# speedup vs baseline: 4.1778x; 4.1778x over previous
import functools

import jax
import jax.numpy as jnp
import numpy as np
from jax import lax
from jax.experimental import pallas as pl
from jax.experimental.pallas import tpu as pltpu
from jax.experimental.pallas import tpu_sc as plsc

_EPS = 1e-5
_K = 16
_B = 2
_N = 1024
_BN = _B * _N
_M = _BN * _K
_D1 = 1024
_DS = 128
_DT = _D1 + _DS


def _bn0(t):
    m = jnp.mean(t, axis=0, keepdims=True)
    v = jnp.mean((t - m) ** 2, axis=0, keepdims=True)
    return (t - m) / jnp.sqrt(v + _EPS)


def _relu(t):
    return jnp.maximum(t, 0.0)


def _tnet_ref(x, p):
    def bn(t, axes):
        m = jnp.mean(t, axis=axes, keepdims=True)
        v = jnp.var(t, axis=axes, keepdims=True)
        return (t - m) / jnp.sqrt(v + _EPS)

    def c1(t, w, b):
        return jnp.einsum('bcn,oc->bon', t, w) + b[None, :, None]

    xyz = x[:, :3, :]
    t = jax.nn.relu(bn(c1(xyz, p['t_c1_w'], p['t_c1_b']), (0, 2)))
    t = jax.nn.relu(bn(c1(t, p['t_c2_w'], p['t_c2_b']), (0, 2)))
    t = jax.nn.relu(bn(c1(t, p['t_c3_w'], p['t_c3_b']), (0, 2)))
    t = jnp.max(t, axis=2)
    t = jax.nn.relu(bn(t @ p['t_f1_w'].T + p['t_f1_b'], (0,)))
    t = jax.nn.relu(bn(t @ p['t_f2_w'].T + p['t_f2_b'], (0,)))
    t = t @ p['t_f3_w'].T + p['t_f3_b']
    t = t + jnp.eye(3, dtype=t.dtype).reshape(1, 9)
    trans = t.reshape(-1, 3, 3)
    return jnp.einsum('bij,bjn->bin', trans, xyz)


def _a2_body(xt_ref, xtr_ref, xyzn_ref,
             c1w, c1b, c2w, c2b, c3w, c3b, c4w, c4b,
             wn1, wx1, wc1, e1b, wns, wxs, wcs, esb,
             table_ref, cen1_ref, cens_ref):
    def extract(v):
        f1 = _relu(_bn0(jnp.dot(v, c1w[...]) + c1b[...]))
        f2 = _relu(_bn0(jnp.dot(f1, c2w[...]) + c2b[...]))
        f3 = _relu(_bn0(jnp.dot(f2, c3w[...]) + c3b[...]))
        f4 = _relu(_bn0(jnp.dot(f3, c4w[...]) + c4b[...]))
        return jnp.concatenate([f1, f2, f3, f4], axis=1)

    feat_orig = extract(xt_ref[...])
    feat_trans = extract(xtr_ref[...])
    xyzn = xyzn_ref[...]
    qx1 = jnp.dot(xyzn, wx1[...])
    qxs = jnp.dot(xyzn, wxs[...])
    table_ref[:, :_D1] = jnp.dot(feat_trans, wn1[...]) + qx1
    table_ref[:, _D1:] = jnp.dot(feat_trans, wns[...]) + qxs
    cen1_ref[...] = jnp.dot(feat_orig, wc1[...]) - qx1 + e1b[...]
    cens_ref[...] = jnp.dot(feat_orig, wcs[...]) - qxs + esb[...]


def _a3_body(xyzn_ref, idx_ref):
    big = jnp.float32(1e30)
    for b in range(_B):
        xb = xyzn_ref[b * _N:(b + 1) * _N, :]
        sq = jnp.sum(xb * xb, axis=1, keepdims=True)
        d = -2.0 * lax.dot_general(xb, xb, (((1,), (1,)), ((), ())))
        d = d + sq + jnp.transpose(sq)
        iota = lax.broadcasted_iota(jnp.int32, (_N, _N), 1)
        for k in range(_K):
            m = jnp.min(d, axis=1, keepdims=True)
            sel = d == m
            ik = jnp.min(jnp.where(sel, iota, _N), axis=1)
            d = jnp.where(iota == ik[:, None], big, d)
            idx_ref[b, k, :] = ik + b * _N


def _sc_gather(table, idx_flat):
    mesh = plsc.VectorSubcoreMesh(core_axis_name="c", subcore_axis_name="s")
    nw = 32
    ch = 64
    per_w = _M // nw

    @functools.partial(
        pl.kernel, mesh=mesh,
        out_type=jax.ShapeDtypeStruct((_M, _DT), jnp.float32),
        scratch_types=[
            pltpu.VMEM((ch,), jnp.int32),
            pltpu.VMEM((ch, _DT), jnp.float32),
            pltpu.SemaphoreType.DMA,
        ],
    )
    def gk(table_hbm, idx_hbm, out_hbm, idx_v, rows_v, sem):
        wid = lax.axis_index("s") * 2 + lax.axis_index("c")
        for i in range(per_w // ch):
            base = wid * per_w + i * ch
            pltpu.sync_copy(idx_hbm.at[pl.ds(base, ch)], idx_v)
            pltpu.async_copy(table_hbm.at[idx_v], rows_v, sem).wait()
            pltpu.sync_copy(rows_v, out_hbm.at[pl.ds(base, ch)])

    return gk(table, idx_flat)


def _c_body(g_ref, c1_ref, cs_ref, st1_ref, sts_ref):
    g = pl.program_id(0)

    @pl.when(g == 0)
    def _():
        st1_ref[...] = jnp.zeros_like(st1_ref)
        sts_ref[...] = jnp.zeros_like(sts_ref)

    gb = g_ref[...].reshape(128, _K, _DT)
    z1 = (gb[:, :, :_D1] + c1_ref[...][:, None, :]).reshape(128 * _K, _D1)
    zs = (gb[:, :, _D1:] + cs_ref[...][:, None, :]).reshape(128 * _K, _DS)
    st1_ref[0:1, :] += jnp.sum(z1, axis=0, keepdims=True)
    st1_ref[1:2, :] += jnp.sum(z1 * z1, axis=0, keepdims=True)
    sts_ref[0:1, :] += jnp.sum(zs, axis=0, keepdims=True)
    sts_ref[1:2, :] += jnp.sum(zs * zs, axis=0, keepdims=True)


def _d_body(g_ref, c1_ref, cs_ref, st1_ref, sts_ref, e2w, e2b, esr,
            z2_ref, lg_ref):
    mf = jnp.float32(_M)
    mu1 = st1_ref[0:1, :] / mf
    v1 = st1_ref[1:2, :] / mf - mu1 * mu1
    inv1 = 1.0 / jnp.sqrt(v1 + _EPS)
    mus = sts_ref[0:1, :] / mf
    vs = sts_ref[1:2, :] / mf - mus * mus
    invs = 1.0 / jnp.sqrt(vs + _EPS)

    gb = g_ref[...].reshape(128, _K, _DT)
    z1 = (gb[:, :, :_D1] + c1_ref[...][:, None, :]).reshape(128 * _K, _D1)
    zs = (gb[:, :, _D1:] + cs_ref[...][:, None, :]).reshape(128 * _K, _DS)
    h1 = _relu((z1 - mu1) * inv1)
    z2_ref[...] = jnp.dot(h1, e2w[...]) + e2b[...]
    hs = _relu((zs - mus) * invs)
    lg_ref[0] = jnp.sum(hs * esr[...], axis=1, keepdims=True)


def _e_body(z2_ref, lg_ref, xyzn_ref, e3w, e3b, g1w, g1b, g2w, g2b, pw, pb,
            out_ref):
    h2 = _relu(_bn0(z2_ref[...]))
    z3 = jnp.dot(h2, e3w[...]) + e3b[...]
    lf3 = _relu(_bn0(z3))
    lg = lg_ref[...]
    lg = lg - jnp.max(lg, axis=1, keepdims=True)
    ex = jnp.exp(lg)
    w = ex / jnp.sum(ex, axis=1, keepdims=True)
    lf3r = lf3.reshape(_BN, _K, 64)
    local = jnp.sum(lf3r * w[:, :, None], axis=1)
    g = _relu(_bn0(jnp.dot(local, g1w[...]) + g1b[...]))
    gate = jax.nn.sigmoid(jnp.dot(g, g2w[...]) + g2b[...])
    local = local * gate
    tops = []
    for b in range(_B):
        xb = xyzn_ref[b * _N:(b + 1) * _N, :]
        al = lax.dot_general(xb, xb, (((1,), (1,)), ((), ())))
        al = al - jnp.max(al, axis=1, keepdims=True)
        ea = jnp.exp(al)
        aff = ea / jnp.sum(ea, axis=1, keepdims=True)
        lb = local[b * _N:(b + 1) * _N, :]
        tops.append(lax.dot_general(aff, lb, (((0,), (0,)), ((), ()))))
    topo = jnp.concatenate(tops, axis=0)
    out_ref[...] = _relu(_bn0(jnp.dot(topo, pw[...]) + pb[...]))


def _wT(p, name):
    return jnp.transpose(p[name]).astype(jnp.float32)


def _bR(p, name):
    return p[name].reshape(1, -1).astype(jnp.float32)


def _stage_a2(x_t, x_trans, xyz_new, p):
    f32 = jnp.float32
    wT = functools.partial(_wT, p)
    bR = functools.partial(_bR, p)
    e1w, es1w = p['e1_w'], p['es1_w']
    return pl.pallas_call(
        _a2_body,
        out_shape=(
            jax.ShapeDtypeStruct((_BN, _DT), f32),
            jax.ShapeDtypeStruct((_BN, _D1), f32),
            jax.ShapeDtypeStruct((_BN, _DS), f32),
        ),
    )(x_t, x_trans, xyz_new,
      wT('c1_w'), bR('c1_b'), wT('c2_w'), bR('c2_b'),
      wT('c3_w'), bR('c3_b'), wT('c4_w'), bR('c4_b'),
      jnp.transpose(e1w[:, 960:1920]), jnp.transpose(e1w[:, 1920:]),
      jnp.transpose(e1w[:, :960]), bR('e1_b'),
      jnp.transpose(es1w[:, 960:1920]), jnp.transpose(es1w[:, 1920:]),
      jnp.transpose(es1w[:, :960]), bR('es1_b'))


def _stage_a3(xyz_new):
    idx = pl.pallas_call(
        _a3_body,
        out_shape=jax.ShapeDtypeStruct((_B, _K, _N), jnp.int32),
    )(xyz_new)
    return jnp.transpose(idx, (0, 2, 1)).reshape(_M)


_NTILE = _M // 2048
_GRID_CP = pltpu.CompilerParams(dimension_semantics=("arbitrary",))


def _stage_c(gathered, cen1, cens):
    f32 = jnp.float32
    ntile = _NTILE
    grid_cp = _GRID_CP
    return pl.pallas_call(
        _c_body,
        grid=(ntile,),
        in_specs=[
            pl.BlockSpec((2048, _DT), lambda g: (g, 0)),
            pl.BlockSpec((128, _D1), lambda g: (g, 0)),
            pl.BlockSpec((128, _DS), lambda g: (g, 0)),
        ],
        out_specs=(
            pl.BlockSpec((8, _D1), lambda g: (0, 0)),
            pl.BlockSpec((8, _DS), lambda g: (0, 0)),
        ),
        out_shape=(
            jax.ShapeDtypeStruct((8, _D1), f32),
            jax.ShapeDtypeStruct((8, _DS), f32),
        ),
        compiler_params=grid_cp,
    )(gathered, cen1, cens)


def _stage_d(gathered, cen1, cens, st1, sts, p):
    f32 = jnp.float32
    ntile = _NTILE
    grid_cp = _GRID_CP
    wT = functools.partial(_wT, p)
    bR = functools.partial(_bR, p)
    z2, lg3 = pl.pallas_call(
        _d_body,
        grid=(ntile,),
        in_specs=[
            pl.BlockSpec((2048, _DT), lambda g: (g, 0)),
            pl.BlockSpec((128, _D1), lambda g: (g, 0)),
            pl.BlockSpec((128, _DS), lambda g: (g, 0)),
            pl.BlockSpec((8, _D1), lambda g: (0, 0)),
            pl.BlockSpec((8, _DS), lambda g: (0, 0)),
            pl.BlockSpec((_D1, _DS), lambda g: (0, 0)),
            pl.BlockSpec((1, _DS), lambda g: (0, 0)),
            pl.BlockSpec((1, _DS), lambda g: (0, 0)),
        ],
        out_specs=(
            pl.BlockSpec((2048, _DS), lambda g: (g, 0)),
            pl.BlockSpec((1, 2048, 1), lambda g: (g, 0, 0)),
        ),
        out_shape=(
            jax.ShapeDtypeStruct((_M, _DS), f32),
            jax.ShapeDtypeStruct((ntile, 2048, 1), f32),
        ),
        compiler_params=grid_cp,
    )(gathered, cen1, cens, st1, sts,
      wT('e2_w'), bR('e2_b'), bR('es2_w'))
    return z2, lg3.reshape(_BN, _K)


def _stage_e(z2, logits, xyz_new, p):
    f32 = jnp.float32
    wT = functools.partial(_wT, p)
    bR = functools.partial(_bR, p)
    return pl.pallas_call(
        _e_body,
        out_shape=jax.ShapeDtypeStruct((_BN, 64), f32),
    )(z2, logits, xyz_new, wT('e3_w'), bR('e3_b'), wT('g1_w'), bR('g1_b'),
      wT('g2_w'), bR('g2_b'), wT('proj_w'), bR('proj_b'))


def kernel(x, params):
    p = params
    x_t = jnp.transpose(x, (0, 2, 1)).reshape(_BN, 9).astype(jnp.float32)
    xyz_trans = _tnet_ref(x, p)
    xyz_new = jnp.transpose(xyz_trans, (0, 2, 1)).reshape(_BN, 3)
    x_trans = jnp.concatenate([xyz_new, x_t[:, 3:]], axis=1)
    table, cen1, cens = _stage_a2(x_t, x_trans, xyz_new, p)
    idx_flat = _stage_a3(xyz_new)
    gathered = _sc_gather(table, idx_flat)
    st1, sts = _stage_c(gathered, cen1, cens)
    z2, logits = _stage_d(gathered, cen1, cens, st1, sts, p)
    out = _stage_e(z2, logits, xyz_new, p)
    topo_feat = jnp.transpose(out.reshape(_B, _N, 64), (0, 2, 1))
    return (xyz_trans, topo_feat)


# speedup vs baseline: 4.2462x; 1.0164x over previous
import functools

import jax
import jax.numpy as jnp
import numpy as np
from jax import lax
from jax.experimental import pallas as pl
from jax.experimental.pallas import tpu as pltpu
from jax.experimental.pallas import tpu_sc as plsc

_EPS = 1e-5
_K = 16
_B = 2
_N = 1024
_BN = _B * _N
_M = _BN * _K
_D1 = 1024
_DS = 128
_DT = _D1 + _DS


def _bn0(t):
    m = jnp.mean(t, axis=0, keepdims=True)
    v = jnp.mean((t - m) ** 2, axis=0, keepdims=True)
    return (t - m) / jnp.sqrt(v + _EPS)


def _relu(t):
    return jnp.maximum(t, 0.0)


def _tnet_ref(x, p):
    def bn(t, axes):
        m = jnp.mean(t, axis=axes, keepdims=True)
        v = jnp.var(t, axis=axes, keepdims=True)
        return (t - m) / jnp.sqrt(v + _EPS)

    def c1(t, w, b):
        return jnp.einsum('bcn,oc->bon', t, w) + b[None, :, None]

    xyz = x[:, :3, :]
    t = jax.nn.relu(bn(c1(xyz, p['t_c1_w'], p['t_c1_b']), (0, 2)))
    t = jax.nn.relu(bn(c1(t, p['t_c2_w'], p['t_c2_b']), (0, 2)))
    t = jax.nn.relu(bn(c1(t, p['t_c3_w'], p['t_c3_b']), (0, 2)))
    t = jnp.max(t, axis=2)
    t = jax.nn.relu(bn(t @ p['t_f1_w'].T + p['t_f1_b'], (0,)))
    t = jax.nn.relu(bn(t @ p['t_f2_w'].T + p['t_f2_b'], (0,)))
    t = t @ p['t_f3_w'].T + p['t_f3_b']
    t = t + jnp.eye(3, dtype=t.dtype).reshape(1, 9)
    trans = t.reshape(-1, 3, 3)
    return jnp.einsum('bij,bjn->bin', trans, xyz)


def _a2_body(xt_ref, xtr_ref, xyzn_ref,
             c1w, c1b, c2w, c2b, c3w, c3b, c4w, c4b,
             wn1, wx1, wc1, e1b, wns, wxs, wcs, esb,
             table_ref, cen1_ref, cens_ref):
    def extract(v):
        f1 = _relu(_bn0(jnp.dot(v, c1w[...]) + c1b[...]))
        f2 = _relu(_bn0(jnp.dot(f1, c2w[...]) + c2b[...]))
        f3 = _relu(_bn0(jnp.dot(f2, c3w[...]) + c3b[...]))
        f4 = _relu(_bn0(jnp.dot(f3, c4w[...]) + c4b[...]))
        return jnp.concatenate([f1, f2, f3, f4], axis=1)

    feat_orig = extract(xt_ref[...])
    feat_trans = extract(xtr_ref[...])
    xyzn = xyzn_ref[...]
    qx1 = jnp.dot(xyzn, wx1[...])
    qxs = jnp.dot(xyzn, wxs[...])
    table_ref[:, :_D1] = jnp.dot(feat_trans, wn1[...]) + qx1
    table_ref[:, _D1:] = jnp.dot(feat_trans, wns[...]) + qxs
    cen1_ref[...] = jnp.dot(feat_orig, wc1[...]) - qx1 + e1b[...]
    cens_ref[...] = jnp.dot(feat_orig, wcs[...]) - qxs + esb[...]


def _a3_body(xyzn_ref, idx_ref):
    big = jnp.float32(1e30)
    for b in range(_B):
        xb = xyzn_ref[b * _N:(b + 1) * _N, :]
        sq = jnp.sum(xb * xb, axis=1, keepdims=True)
        d = -2.0 * lax.dot_general(xb, xb, (((1,), (1,)), ((), ())))
        d = d + sq + jnp.transpose(sq)
        iota = lax.broadcasted_iota(jnp.int32, (_N, _N), 1)
        for k in range(_K):
            m = jnp.min(d, axis=1, keepdims=True)
            sel = d == m
            ik = jnp.min(jnp.where(sel, iota, _N), axis=1)
            d = jnp.where(iota == ik[:, None], big, d)
            idx_ref[b, k, :] = ik + b * _N


def _sc_gather(table, idx_flat):
    mesh = plsc.VectorSubcoreMesh(core_axis_name="c", subcore_axis_name="s")
    nw = 32
    ch = 32
    per_w = _M // nw
    nch = per_w // ch

    @functools.partial(
        pl.kernel, mesh=mesh,
        out_type=jax.ShapeDtypeStruct((_M, _DT), jnp.float32),
        scratch_types=[
            pltpu.VMEM((2, ch), jnp.int32),
            pltpu.VMEM((2, ch, _DT), jnp.float32),
            pltpu.SemaphoreType.DMA((2,)),
            pltpu.SemaphoreType.DMA((2,)),
        ],
    )
    def gk(table_hbm, idx_hbm, out_hbm, idx_v, rows_v, gsem, wsem):
        wid = lax.axis_index("s") * 2 + lax.axis_index("c")

        def start_gather(i, slot):
            base = wid * per_w + i * ch
            pltpu.sync_copy(idx_hbm.at[pl.ds(base, ch)], idx_v.at[slot])
            pltpu.make_async_copy(
                table_hbm.at[idx_v.at[slot]], rows_v.at[slot], gsem.at[slot]
            ).start()

        start_gather(0, 0)
        for i in range(nch):
            cur = i % 2
            if i + 1 < nch:
                nxt = (i + 1) % 2
                if i >= 1:
                    pltpu.make_async_copy(
                        rows_v.at[nxt],
                        out_hbm.at[pl.ds(wid * per_w + (i - 1) * ch, ch)],
                        wsem.at[nxt],
                    ).wait()
                start_gather(i + 1, nxt)
            base = wid * per_w + i * ch
            pltpu.make_async_copy(
                table_hbm.at[idx_v.at[cur]], rows_v.at[cur], gsem.at[cur]
            ).wait()
            pltpu.make_async_copy(
                rows_v.at[cur], out_hbm.at[pl.ds(base, ch)], wsem.at[cur]
            ).start()
        for i in (nch - 2, nch - 1):
            pltpu.make_async_copy(
                rows_v.at[i % 2],
                out_hbm.at[pl.ds(wid * per_w + i * ch, ch)],
                wsem.at[i % 2],
            ).wait()

    return gk(table, idx_flat)


def _c_body(g_ref, c1_ref, cs_ref, st1_ref, sts_ref):
    g = pl.program_id(0)

    @pl.when(g == 0)
    def _():
        st1_ref[...] = jnp.zeros_like(st1_ref)
        sts_ref[...] = jnp.zeros_like(sts_ref)

    gb = g_ref[...].reshape(128, _K, _DT)
    z1 = (gb[:, :, :_D1] + c1_ref[...][:, None, :]).reshape(128 * _K, _D1)
    zs = (gb[:, :, _D1:] + cs_ref[...][:, None, :]).reshape(128 * _K, _DS)
    st1_ref[0:1, :] += jnp.sum(z1, axis=0, keepdims=True)
    st1_ref[1:2, :] += jnp.sum(z1 * z1, axis=0, keepdims=True)
    sts_ref[0:1, :] += jnp.sum(zs, axis=0, keepdims=True)
    sts_ref[1:2, :] += jnp.sum(zs * zs, axis=0, keepdims=True)


def _d_body(g_ref, c1_ref, cs_ref, st1_ref, sts_ref, e2w, e2b, esr,
            z2_ref, lg_ref):
    mf = jnp.float32(_M)
    mu1 = st1_ref[0:1, :] / mf
    v1 = st1_ref[1:2, :] / mf - mu1 * mu1
    inv1 = 1.0 / jnp.sqrt(v1 + _EPS)
    mus = sts_ref[0:1, :] / mf
    vs = sts_ref[1:2, :] / mf - mus * mus
    invs = 1.0 / jnp.sqrt(vs + _EPS)

    gb = g_ref[...].reshape(128, _K, _DT)
    z1 = (gb[:, :, :_D1] + c1_ref[...][:, None, :]).reshape(128 * _K, _D1)
    zs = (gb[:, :, _D1:] + cs_ref[...][:, None, :]).reshape(128 * _K, _DS)
    h1 = _relu((z1 - mu1) * inv1)
    z2_ref[...] = jnp.dot(h1, e2w[...]) + e2b[...]
    hs = _relu((zs - mus) * invs)
    lg_ref[0] = jnp.sum(hs * esr[...], axis=1, keepdims=True)


def _e_body(z2_ref, lg_ref, xyzn_ref, e3w, e3b, g1w, g1b, g2w, g2b, pw, pb,
            out_ref):
    h2 = _relu(_bn0(z2_ref[...]))
    z3 = jnp.dot(h2, e3w[...]) + e3b[...]
    lf3 = _relu(_bn0(z3))
    lg = lg_ref[...]
    lg = lg - jnp.max(lg, axis=1, keepdims=True)
    ex = jnp.exp(lg)
    w = ex / jnp.sum(ex, axis=1, keepdims=True)
    lf3r = lf3.reshape(_BN, _K, 64)
    local = jnp.sum(lf3r * w[:, :, None], axis=1)
    g = _relu(_bn0(jnp.dot(local, g1w[...]) + g1b[...]))
    gate = jax.nn.sigmoid(jnp.dot(g, g2w[...]) + g2b[...])
    local = local * gate
    tops = []
    for b in range(_B):
        xb = xyzn_ref[b * _N:(b + 1) * _N, :]
        al = lax.dot_general(xb, xb, (((1,), (1,)), ((), ())))
        al = al - jnp.max(al, axis=1, keepdims=True)
        ea = jnp.exp(al)
        aff = ea / jnp.sum(ea, axis=1, keepdims=True)
        lb = local[b * _N:(b + 1) * _N, :]
        tops.append(lax.dot_general(aff, lb, (((0,), (0,)), ((), ()))))
    topo = jnp.concatenate(tops, axis=0)
    out_ref[...] = _relu(_bn0(jnp.dot(topo, pw[...]) + pb[...]))


def _wT(p, name):
    return jnp.transpose(p[name]).astype(jnp.float32)


def _bR(p, name):
    return p[name].reshape(1, -1).astype(jnp.float32)


def _stage_a2(x_t, x_trans, xyz_new, p):
    f32 = jnp.float32
    wT = functools.partial(_wT, p)
    bR = functools.partial(_bR, p)
    e1w, es1w = p['e1_w'], p['es1_w']
    return pl.pallas_call(
        _a2_body,
        out_shape=(
            jax.ShapeDtypeStruct((_BN, _DT), f32),
            jax.ShapeDtypeStruct((_BN, _D1), f32),
            jax.ShapeDtypeStruct((_BN, _DS), f32),
        ),
    )(x_t, x_trans, xyz_new,
      wT('c1_w'), bR('c1_b'), wT('c2_w'), bR('c2_b'),
      wT('c3_w'), bR('c3_b'), wT('c4_w'), bR('c4_b'),
      jnp.transpose(e1w[:, 960:1920]), jnp.transpose(e1w[:, 1920:]),
      jnp.transpose(e1w[:, :960]), bR('e1_b'),
      jnp.transpose(es1w[:, 960:1920]), jnp.transpose(es1w[:, 1920:]),
      jnp.transpose(es1w[:, :960]), bR('es1_b'))


def _stage_a3(xyz_new):
    idx = pl.pallas_call(
        _a3_body,
        out_shape=jax.ShapeDtypeStruct((_B, _K, _N), jnp.int32),
    )(xyz_new)
    return jnp.transpose(idx, (0, 2, 1)).reshape(_M)


_NTILE = _M // 2048
_GRID_CP = pltpu.CompilerParams(dimension_semantics=("arbitrary",))


def _stage_c(gathered, cen1, cens):
    f32 = jnp.float32
    ntile = _NTILE
    grid_cp = _GRID_CP
    return pl.pallas_call(
        _c_body,
        grid=(ntile,),
        in_specs=[
            pl.BlockSpec((2048, _DT), lambda g: (g, 0)),
            pl.BlockSpec((128, _D1), lambda g: (g, 0)),
            pl.BlockSpec((128, _DS), lambda g: (g, 0)),
        ],
        out_specs=(
            pl.BlockSpec((8, _D1), lambda g: (0, 0)),
            pl.BlockSpec((8, _DS), lambda g: (0, 0)),
        ),
        out_shape=(
            jax.ShapeDtypeStruct((8, _D1), f32),
            jax.ShapeDtypeStruct((8, _DS), f32),
        ),
        compiler_params=grid_cp,
    )(gathered, cen1, cens)


def _stage_d(gathered, cen1, cens, st1, sts, p):
    f32 = jnp.float32
    ntile = _NTILE
    grid_cp = _GRID_CP
    wT = functools.partial(_wT, p)
    bR = functools.partial(_bR, p)
    z2, lg3 = pl.pallas_call(
        _d_body,
        grid=(ntile,),
        in_specs=[
            pl.BlockSpec((2048, _DT), lambda g: (g, 0)),
            pl.BlockSpec((128, _D1), lambda g: (g, 0)),
            pl.BlockSpec((128, _DS), lambda g: (g, 0)),
            pl.BlockSpec((8, _D1), lambda g: (0, 0)),
            pl.BlockSpec((8, _DS), lambda g: (0, 0)),
            pl.BlockSpec((_D1, _DS), lambda g: (0, 0)),
            pl.BlockSpec((1, _DS), lambda g: (0, 0)),
            pl.BlockSpec((1, _DS), lambda g: (0, 0)),
        ],
        out_specs=(
            pl.BlockSpec((2048, _DS), lambda g: (g, 0)),
            pl.BlockSpec((1, 2048, 1), lambda g: (g, 0, 0)),
        ),
        out_shape=(
            jax.ShapeDtypeStruct((_M, _DS), f32),
            jax.ShapeDtypeStruct((ntile, 2048, 1), f32),
        ),
        compiler_params=grid_cp,
    )(gathered, cen1, cens, st1, sts,
      wT('e2_w'), bR('e2_b'), bR('es2_w'))
    return z2, lg3.reshape(_BN, _K)


def _stage_e(z2, logits, xyz_new, p):
    f32 = jnp.float32
    wT = functools.partial(_wT, p)
    bR = functools.partial(_bR, p)
    return pl.pallas_call(
        _e_body,
        out_shape=jax.ShapeDtypeStruct((_BN, 64), f32),
    )(z2, logits, xyz_new, wT('e3_w'), bR('e3_b'), wT('g1_w'), bR('g1_b'),
      wT('g2_w'), bR('g2_b'), wT('proj_w'), bR('proj_b'))


def kernel(x, params):
    p = params
    x_t = jnp.transpose(x, (0, 2, 1)).reshape(_BN, 9).astype(jnp.float32)
    xyz_trans = _tnet_ref(x, p)
    xyz_new = jnp.transpose(xyz_trans, (0, 2, 1)).reshape(_BN, 3)
    x_trans = jnp.concatenate([xyz_new, x_t[:, 3:]], axis=1)
    table, cen1, cens = _stage_a2(x_t, x_trans, xyz_new, p)
    idx_flat = _stage_a3(xyz_new)
    gathered = _sc_gather(table, idx_flat)
    st1, sts = _stage_c(gathered, cen1, cens)
    z2, logits = _stage_d(gathered, cen1, cens, st1, sts, p)
    out = _stage_e(z2, logits, xyz_new, p)
    topo_feat = jnp.transpose(out.reshape(_B, _N, 64), (0, 2, 1))
    return (xyz_trans, topo_feat)
